# baseline (speedup 1.0000x reference)
import jax
import jax.numpy as jnp
from jax.experimental import pallas as pl
from jax.experimental.pallas import tpu as pltpu

NFEAT = 128
NHID = 32
NCLASS = 16
NHEADS = 8
ALPHA = 0.2
B = 4
N = 1024
HEXT = NHID + 1


def _gat_kernel(x_ref, w_ref, amat_ref, fc_ref, out_ref):
    x = x_ref[0]
    w = w_ref[0]
    amat = amat_ref[0]
    hm_all = jnp.dot(x, w, preferred_element_type=jnp.float32)
    f_all = jnp.dot(hm_all, amat, preferred_element_type=jnp.float32)

    acc = jnp.zeros((1, NCLASS), jnp.float32)
    for h in range(NHEADS):
        hm_ext = hm_all[:, h * HEXT:(h + 1) * HEXT]
        f1 = f_all[:, 2 * h:2 * h + 1]
        f2 = f_all[:, 2 * h + 1:2 * h + 2]
        mx = jnp.max(f2)
        m = f1 + mx
        m = jnp.where(m > 0, m, ALPHA * m)
        g1 = f1 - m
        g1a = ALPHA * f1 - m
        g2 = f2.T
        g2a = ALPHA * g2
        p = jnp.exp(jnp.maximum(g1 + g2, g1a + g2a))
        num = jnp.dot(p, hm_ext, preferred_element_type=jnp.float32)
        hp = num[:, :NHID] / num[:, NHID:]
        hp = jnp.where(hp > 0, hp, jnp.exp(jnp.minimum(hp, 0.0)) - 1.0)
        pooled = jnp.max(hp, axis=0, keepdims=True)
        acc = acc + jnp.dot(pooled, fc_ref[h], preferred_element_type=jnp.float32)
    out_ref[0] = acc


def kernel(x, W, a, FC):
    x_ext = jnp.concatenate([x, jnp.ones((B, N, 1), jnp.float32)], axis=2)
    w_ext = jnp.zeros((NFEAT + 1, NHEADS * HEXT), jnp.float32)
    for h in range(NHEADS):
        w_ext = w_ext.at[:NFEAT, h * HEXT:h * HEXT + NHID].set(W[h])
        w_ext = w_ext.at[NFEAT, h * HEXT + NHID].set(1.0)
    amat = jnp.zeros((NHEADS * HEXT, 2 * NHEADS), jnp.float32)
    for h in range(NHEADS):
        amat = amat.at[h * HEXT:h * HEXT + NHID, 2 * h].set(a[h, :NHID, 0])
        amat = amat.at[h * HEXT:h * HEXT + NHID, 2 * h + 1].set(a[h, NHID:, 0])
    fc3d = FC.reshape(NHEADS, NHID, NCLASS)

    out = pl.pallas_call(
        _gat_kernel,
        grid=(B,),
        in_specs=[
            pl.BlockSpec((1, N, NFEAT + 1), lambda b: (b, 0, 0)),
            pl.BlockSpec((1, NFEAT + 1, NHEADS * HEXT), lambda b: (0, 0, 0)),
            pl.BlockSpec((1, NHEADS * HEXT, 2 * NHEADS), lambda b: (0, 0, 0)),
            pl.BlockSpec((NHEADS, NHID, NCLASS), lambda b: (0, 0, 0)),
        ],
        out_specs=pl.BlockSpec((1, 1, NCLASS), lambda b: (b, 0, 0)),
        out_shape=jax.ShapeDtypeStruct((B, 1, NCLASS), jnp.float32),
        compiler_params=pltpu.CompilerParams(
            dimension_semantics=("parallel",),
        ),
    )(x_ext, w_ext[None], amat[None], fc3d)
    return out.reshape(B, NCLASS)


# speedup vs baseline: 1.7261x; 1.7261x over previous
import jax
import jax.numpy as jnp
from jax.experimental import pallas as pl
from jax.experimental.pallas import tpu as pltpu

NFEAT = 128
NHID = 32
NCLASS = 16
NHEADS = 8
ALPHA = 0.2
B = 4
N = 1024


def _gat_kernel(x_ref, xt_ref, w_ref, c1_ref, c2_ref, fc_ref, out_ref):
    h_idx = pl.program_id(1)

    x = x_ref[0]
    xt = xt_ref[0]
    w = w_ref[0]
    hm = jnp.dot(x, w, preferred_element_type=jnp.float32)

    f1 = jnp.dot(x, c1_ref[0], preferred_element_type=jnp.float32)
    f2r = jnp.dot(c2_ref[0], xt, preferred_element_type=jnp.float32)

    mx = jnp.max(f2r)
    m = f1 + mx
    m = jnp.where(m > 0, m, ALPHA * m)
    g1 = f1 - m
    g1a = ALPHA * f1 - m
    g2a = ALPHA * f2r
    p = jnp.exp(jnp.maximum(g1 + f2r, g1a + g2a))
    hm_ext = jnp.concatenate([hm, jnp.ones((N, 1), jnp.float32)], axis=1)
    num = jnp.dot(p, hm_ext, preferred_element_type=jnp.float32)
    hp = num[:, :NHID] / num[:, NHID:]
    hp = jnp.where(hp > 0, hp, jnp.exp(jnp.minimum(hp, 0.0)) - 1.0)
    pooled = jnp.max(hp, axis=0, keepdims=True)

    contrib = jnp.dot(pooled, fc_ref[0], preferred_element_type=jnp.float32)

    @pl.when(h_idx == 0)
    def _():
        out_ref[0] = contrib

    @pl.when(h_idx != 0)
    def _():
        out_ref[0] += contrib


def kernel(x, W, a, FC):
    c1 = jnp.einsum('hfo,ho->hf', W, a[:, :NHID, 0])[:, :, None]
    c2 = jnp.einsum('hfo,ho->hf', W, a[:, NHID:, 0])[:, None, :]
    xt = jnp.swapaxes(x, 1, 2)
    fc3d = FC.reshape(NHEADS, NHID, NCLASS)

    out = pl.pallas_call(
        _gat_kernel,
        grid=(B, NHEADS),
        in_specs=[
            pl.BlockSpec((1, N, NFEAT), lambda b, h: (b, 0, 0)),
            pl.BlockSpec((1, NFEAT, N), lambda b, h: (b, 0, 0)),
            pl.BlockSpec((1, NFEAT, NHID), lambda b, h: (h, 0, 0)),
            pl.BlockSpec((1, NFEAT, 1), lambda b, h: (h, 0, 0)),
            pl.BlockSpec((1, 1, NFEAT), lambda b, h: (h, 0, 0)),
            pl.BlockSpec((1, NHID, NCLASS), lambda b, h: (h, 0, 0)),
        ],
        out_specs=pl.BlockSpec((1, 1, NCLASS), lambda b, h: (b, 0, 0)),
        out_shape=jax.ShapeDtypeStruct((B, 1, NCLASS), jnp.float32),
        compiler_params=pltpu.CompilerParams(
            dimension_semantics=("parallel", "arbitrary"),
        ),
    )(x, xt, W, c1, c2, fc3d)
    return out.reshape(B, NCLASS)
